# baseline (speedup 1.0000x reference)
import functools
import jax
import jax.numpy as jnp
from jax import lax
from jax.experimental import pallas as pl
from jax.experimental.pallas import tpu as pltpu
from jax.experimental.pallas import tpu_sc as plsc

N = 10000
E = 320000
D = 128

NC = 2
NS = 16
DH = D // NC
EPT = E // NS
CH = 125
NCHUNK = EPT // CH
NBUF = 4
NP = 10240
RPT = NP // NS

_sc_mesh = plsc.VectorSubcoreMesh(core_axis_name="c", subcore_axis_name="s")


@functools.partial(
    pl.kernel,
    mesh=_sc_mesh,
    compiler_params=pltpu.CompilerParams(use_tc_tiling_on_sc=False),
    out_type=jax.ShapeDtypeStruct((NP, D), jnp.float32),
    scratch_types=[
        pltpu.VMEM((NCHUNK, CH), jnp.int32),
        pltpu.VMEM((NCHUNK, CH), jnp.int32),
        [pltpu.VMEM((CH, DH), jnp.float32)] * NBUF,
        pltpu.VMEM_SHARED((NP, DH), jnp.float32),
        [pltpu.SemaphoreType.DMA] * NBUF,
        [pltpu.SemaphoreType.DMA] * NBUF,
    ],
)
def _sc_aggregate(xl_hbm, srce_hbm, srco_hbm, dst_hbm, zeros_hbm, out_hbm,
                  src_v, dst_v, bufs, agg_sh, gsems, ssems):
    cid = lax.axis_index("c")
    sid = lax.axis_index("s")

    pltpu.sync_copy(zeros_hbm, agg_sh.at[pl.ds(sid * RPT, RPT)])

    @pl.when(cid == 0)
    def _():
        pltpu.sync_copy(srce_hbm.at[sid], src_v)

    @pl.when(cid == 1)
    def _():
        pltpu.sync_copy(srco_hbm.at[sid], src_v)

    pltpu.sync_copy(dst_hbm.at[sid], dst_v)

    plsc.subcore_barrier()

    def gather(j, b):
        pltpu.async_copy(xl_hbm.at[src_v.at[j]], bufs[b], gsems[b])

    def gwait(b):
        pltpu.make_async_copy(xl_hbm.at[src_v.at[0]], bufs[b],
                              gsems[b]).wait()

    def scatter(j, b):
        pltpu.async_copy(bufs[b], agg_sh.at[dst_v.at[j]], ssems[b], add=True)

    def swait(b):
        pltpu.make_async_copy(bufs[b], agg_sh.at[dst_v.at[0]],
                              ssems[b]).wait()

    for b in range(NBUF):
        gather(b, b)

    def body(r, carry):
        j = r * NBUF
        for b in range(NBUF):
            gwait(b)
            scatter(j + b, b)
        for b in range(NBUF):
            swait(b)
            gather(j + NBUF + b, b)
        return carry

    lax.fori_loop(0, NCHUNK // NBUF - 1, body, 0)

    j_last = NCHUNK - NBUF
    for b in range(NBUF):
        gwait(b)
        scatter(j_last + b, b)
    for b in range(NBUF):
        swait(b)

    plsc.subcore_barrier()

    pltpu.sync_copy(agg_sh.at[pl.ds(sid * RPT, RPT)],
                    out_hbm.at[pl.ds(sid * RPT, RPT), pl.ds(cid * DH, DH)])


def _tc_body(x_ref, agg_ref, w_ref, b_ref, eps_ref, gamma_ref,
             beta_ref, out_ref):
    x = x_ref[...]
    u = (1.0 + eps_ref[0]) * x + agg_ref[:N, :]
    h = lax.dot_general(u, w_ref[...], (((1,), (1,)), ((), ())),
                        preferred_element_type=jnp.float32)
    h = h + b_ref[...]
    mean = jnp.mean(h, axis=0, keepdims=True)
    var = jnp.mean((h - mean) ** 2, axis=0, keepdims=True)
    h = (h - mean) * lax.rsqrt(var + 1e-5) * gamma_ref[...] + beta_ref[...]
    out_ref[...] = jnp.maximum(h, 0.0) + x


_tc_finish = pl.pallas_call(
    _tc_body,
    out_shape=jax.ShapeDtypeStruct((N, D), jnp.float32),
    in_specs=[
        pl.BlockSpec(memory_space=pltpu.VMEM),
        pl.BlockSpec(memory_space=pltpu.VMEM),
        pl.BlockSpec(memory_space=pltpu.VMEM),
        pl.BlockSpec(memory_space=pltpu.VMEM),
        pl.BlockSpec(memory_space=pltpu.SMEM),
        pl.BlockSpec(memory_space=pltpu.VMEM),
        pl.BlockSpec(memory_space=pltpu.VMEM),
    ],
    out_specs=pl.BlockSpec(memory_space=pltpu.VMEM),
)


@jax.jit
def kernel(x, edge_index, W, b, eps, gamma, beta):
    xl = x.reshape(NC * N, DH)
    src = edge_index[0].reshape(NS, NCHUNK, CH)
    dst = edge_index[1].reshape(NS, NCHUNK, CH)
    src_even = src * 2
    src_odd = src * 2 + 1
    zeros = jnp.zeros((RPT, DH), jnp.float32)
    agg = _sc_aggregate(xl, src_even, src_odd, dst, zeros)
    return _tc_finish(x, agg, W, b.reshape(1, D),
                      eps.reshape(1), gamma.reshape(1, D),
                      beta.reshape(1, D))


# speedup vs baseline: 1.0291x; 1.0291x over previous
import functools
import jax
import jax.numpy as jnp
from jax import lax
from jax.experimental import pallas as pl
from jax.experimental.pallas import tpu as pltpu
from jax.experimental.pallas import tpu_sc as plsc

N = 10000
E = 320000
D = 128

NC = 2
NS = 16
DH = D // NC
CH = 128
NCHT = E // CH // NS
NXTRA = E // CH - NCHT * NS
EPT = (NCHT + 1) * CH
NBUF = 4
NP = 10240
RPT = NP // NS

_sc_mesh = plsc.VectorSubcoreMesh(core_axis_name="c", subcore_axis_name="s")


@functools.partial(
    pl.kernel,
    mesh=_sc_mesh,
    compiler_params=pltpu.CompilerParams(use_tc_tiling_on_sc=False),
    out_type=jax.ShapeDtypeStruct((NP, D), jnp.float32),
    scratch_types=[
        pltpu.VMEM((EPT,), jnp.int32),
        pltpu.VMEM((EPT,), jnp.int32),
        [pltpu.VMEM((CH, DH), jnp.float32)] * NBUF,
        pltpu.VMEM_SHARED((NP, DH), jnp.float32),
        [pltpu.SemaphoreType.DMA] * NBUF,
        [pltpu.SemaphoreType.DMA] * NBUF,
    ],
)
def _sc_aggregate(xl_hbm, srce_hbm, srco_hbm, dst_hbm, zeros_hbm, out_hbm,
                  src_v, dst_v, bufs, agg_sh, gsems, ssems):
    cid = lax.axis_index("c")
    sid = lax.axis_index("s")

    pltpu.sync_copy(zeros_hbm, agg_sh.at[pl.ds(sid * RPT, RPT)])

    base = sid * (NCHT * CH)
    main = NCHT * CH

    @pl.when(cid == 0)
    def _():
        pltpu.sync_copy(srce_hbm.at[pl.ds(base, main)],
                        src_v.at[pl.ds(0, main)])

    @pl.when(cid == 1)
    def _():
        pltpu.sync_copy(srco_hbm.at[pl.ds(base, main)],
                        src_v.at[pl.ds(0, main)])

    pltpu.sync_copy(dst_hbm.at[pl.ds(base, main)], dst_v.at[pl.ds(0, main)])

    xbase = NS * (NCHT * CH) + sid * CH

    @pl.when(jnp.logical_and(sid < NXTRA, cid == 0))
    def _():
        pltpu.sync_copy(srce_hbm.at[pl.ds(xbase, CH)],
                        src_v.at[pl.ds(main, CH)])

    @pl.when(jnp.logical_and(sid < NXTRA, cid == 1))
    def _():
        pltpu.sync_copy(srco_hbm.at[pl.ds(xbase, CH)],
                        src_v.at[pl.ds(main, CH)])

    @pl.when(sid < NXTRA)
    def _():
        pltpu.sync_copy(dst_hbm.at[pl.ds(xbase, CH)],
                        dst_v.at[pl.ds(main, CH)])

    plsc.subcore_barrier()

    def gather(j, b):
        pltpu.async_copy(xl_hbm.at[src_v.at[pl.ds(j * CH, CH)]],
                         bufs[b], gsems[b])

    def gwait(b):
        pltpu.make_async_copy(xl_hbm.at[src_v.at[pl.ds(0, CH)]], bufs[b],
                              gsems[b]).wait()

    def scatter(j, b):
        pltpu.async_copy(bufs[b], agg_sh.at[dst_v.at[pl.ds(j * CH, CH)]],
                         ssems[b], add=True)

    def swait(b):
        pltpu.make_async_copy(bufs[b], agg_sh.at[dst_v.at[pl.ds(0, CH)]],
                              ssems[b]).wait()

    for b in range(NBUF):
        gather(b, b)

    def body(r, carry):
        j = r * NBUF
        for b in range(NBUF):
            gwait(b)
            scatter(j + b, b)
        for b in range(NBUF):
            swait(b)
            gather(j + NBUF + b, b)
        return carry

    lax.fori_loop(0, NCHT // NBUF - 1, body, 0)

    j_last = NCHT - NBUF
    for b in range(NBUF):
        gwait(b)
        scatter(j_last + b, b)
    for b in range(NBUF):
        swait(b)

    @pl.when(sid < NXTRA)
    def _():
        gather(NCHT, 0)
        gwait(0)
        scatter(NCHT, 0)
        swait(0)

    plsc.subcore_barrier()

    pltpu.sync_copy(agg_sh.at[pl.ds(sid * RPT, RPT)],
                    out_hbm.at[pl.ds(sid * RPT, RPT), pl.ds(cid * DH, DH)])


def _tc_body(x_ref, agg_ref, w_ref, b_ref, eps_ref, gamma_ref,
             beta_ref, out_ref):
    x = x_ref[...]
    u = (1.0 + eps_ref[0]) * x + agg_ref[:N, :]
    h = lax.dot_general(u, w_ref[...], (((1,), (1,)), ((), ())),
                        preferred_element_type=jnp.float32)
    h = h + b_ref[...]
    mean = jnp.mean(h, axis=0, keepdims=True)
    var = jnp.mean((h - mean) ** 2, axis=0, keepdims=True)
    h = (h - mean) * lax.rsqrt(var + 1e-5) * gamma_ref[...] + beta_ref[...]
    out_ref[...] = jnp.maximum(h, 0.0) + x


_tc_finish = pl.pallas_call(
    _tc_body,
    out_shape=jax.ShapeDtypeStruct((N, D), jnp.float32),
    in_specs=[
        pl.BlockSpec(memory_space=pltpu.VMEM),
        pl.BlockSpec(memory_space=pltpu.VMEM),
        pl.BlockSpec(memory_space=pltpu.VMEM),
        pl.BlockSpec(memory_space=pltpu.VMEM),
        pl.BlockSpec(memory_space=pltpu.SMEM),
        pl.BlockSpec(memory_space=pltpu.VMEM),
        pl.BlockSpec(memory_space=pltpu.VMEM),
    ],
    out_specs=pl.BlockSpec(memory_space=pltpu.VMEM),
)


@jax.jit
def kernel(x, edge_index, W, b, eps, gamma, beta):
    xl = x.reshape(NC * N, DH)
    src = edge_index[0]
    dst = edge_index[1]
    src_even = src * 2
    src_odd = src_even + 1
    zeros = jnp.zeros((RPT, DH), jnp.float32)
    agg = _sc_aggregate(xl, src_even, src_odd, dst, zeros)
    return _tc_finish(x, agg, W, b, eps.reshape(1), gamma, beta)


# speedup vs baseline: 1.0583x; 1.0284x over previous
import functools
import jax
import jax.numpy as jnp
from jax import lax
from jax.experimental import pallas as pl
from jax.experimental.pallas import tpu as pltpu
from jax.experimental.pallas import tpu_sc as plsc

N = 10000
E = 320000
D = 128

NC = 2
NS = 16
DH = D // NC
CH = 128
NCHT = E // CH // NS
NXTRA = E // CH - NCHT * NS
EPT = (NCHT + 1) * CH
NBUF = 6
NP = 10240
RPT = NP // NS

_sc_mesh = plsc.VectorSubcoreMesh(core_axis_name="c", subcore_axis_name="s")


@functools.partial(
    pl.kernel,
    mesh=_sc_mesh,
    compiler_params=pltpu.CompilerParams(use_tc_tiling_on_sc=False),
    out_type=jax.ShapeDtypeStruct((NP, D), jnp.float32),
    scratch_types=[
        pltpu.VMEM((EPT,), jnp.int32),
        pltpu.VMEM((EPT,), jnp.int32),
        [pltpu.VMEM((CH, DH), jnp.float32)] * NBUF,
        pltpu.VMEM_SHARED((NP, DH), jnp.float32),
        [pltpu.SemaphoreType.DMA] * NBUF,
        [pltpu.SemaphoreType.DMA] * NBUF,
    ],
)
def _sc_aggregate(xl_hbm, srce_hbm, srco_hbm, dst_hbm, zeros_hbm, out_hbm,
                  src_v, dst_v, bufs, agg_sh, gsems, ssems):
    cid = lax.axis_index("c")
    sid = lax.axis_index("s")

    pltpu.sync_copy(zeros_hbm, agg_sh.at[pl.ds(sid * RPT, RPT)])

    base = sid * (NCHT * CH)
    main = NCHT * CH

    @pl.when(cid == 0)
    def _():
        pltpu.sync_copy(srce_hbm.at[pl.ds(base, main)],
                        src_v.at[pl.ds(0, main)])

    @pl.when(cid == 1)
    def _():
        pltpu.sync_copy(srco_hbm.at[pl.ds(base, main)],
                        src_v.at[pl.ds(0, main)])

    pltpu.sync_copy(dst_hbm.at[pl.ds(base, main)], dst_v.at[pl.ds(0, main)])

    xbase = NS * (NCHT * CH) + sid * CH

    @pl.when(jnp.logical_and(sid < NXTRA, cid == 0))
    def _():
        pltpu.sync_copy(srce_hbm.at[pl.ds(xbase, CH)],
                        src_v.at[pl.ds(main, CH)])

    @pl.when(jnp.logical_and(sid < NXTRA, cid == 1))
    def _():
        pltpu.sync_copy(srco_hbm.at[pl.ds(xbase, CH)],
                        src_v.at[pl.ds(main, CH)])

    @pl.when(sid < NXTRA)
    def _():
        pltpu.sync_copy(dst_hbm.at[pl.ds(xbase, CH)],
                        dst_v.at[pl.ds(main, CH)])

    plsc.subcore_barrier()

    def gather(j, b):
        pltpu.async_copy(xl_hbm.at[src_v.at[pl.ds(j * CH, CH)]],
                         bufs[b], gsems[b])

    def gwait(b):
        pltpu.make_async_copy(xl_hbm.at[src_v.at[pl.ds(0, CH)]], bufs[b],
                              gsems[b]).wait()

    def scatter(j, b):
        pltpu.async_copy(bufs[b], agg_sh.at[dst_v.at[pl.ds(j * CH, CH)]],
                         ssems[b], add=True)

    def swait(b):
        pltpu.make_async_copy(bufs[b], agg_sh.at[dst_v.at[pl.ds(0, CH)]],
                              ssems[b]).wait()

    for b in range(NBUF):
        gather(b, b)

    def body(r, carry):
        j = r * NBUF
        for b in range(NBUF):
            gwait(b)
            scatter(j + b, b)
        for b in range(NBUF):
            swait(b)
            gather(j + NBUF + b, b)
        return carry

    lax.fori_loop(0, NCHT // NBUF - 1, body, 0)

    j_last = NCHT - NBUF
    for b in range(NBUF):
        gwait(b)
        scatter(j_last + b, b)
    for b in range(NBUF):
        swait(b)

    @pl.when(sid < NXTRA)
    def _():
        gather(NCHT, 0)
        gwait(0)
        scatter(NCHT, 0)
        swait(0)

    plsc.subcore_barrier()

    pltpu.sync_copy(agg_sh.at[pl.ds(sid * RPT, RPT)],
                    out_hbm.at[pl.ds(sid * RPT, RPT), pl.ds(cid * DH, DH)])


def _tc_body(x_ref, agg_ref, w_ref, b_ref, eps_ref, gamma_ref,
             beta_ref, out_ref):
    x = x_ref[...]
    u = (1.0 + eps_ref[0]) * x + agg_ref[:N, :]
    h = lax.dot_general(u, w_ref[...], (((1,), (1,)), ((), ())),
                        preferred_element_type=jnp.float32)
    h = h + b_ref[...]
    mean = jnp.mean(h, axis=0, keepdims=True)
    var = jnp.mean((h - mean) ** 2, axis=0, keepdims=True)
    h = (h - mean) * lax.rsqrt(var + 1e-5) * gamma_ref[...] + beta_ref[...]
    out_ref[...] = jnp.maximum(h, 0.0) + x


_tc_finish = pl.pallas_call(
    _tc_body,
    out_shape=jax.ShapeDtypeStruct((N, D), jnp.float32),
    in_specs=[
        pl.BlockSpec(memory_space=pltpu.VMEM),
        pl.BlockSpec(memory_space=pltpu.VMEM),
        pl.BlockSpec(memory_space=pltpu.VMEM),
        pl.BlockSpec(memory_space=pltpu.VMEM),
        pl.BlockSpec(memory_space=pltpu.SMEM),
        pl.BlockSpec(memory_space=pltpu.VMEM),
        pl.BlockSpec(memory_space=pltpu.VMEM),
    ],
    out_specs=pl.BlockSpec(memory_space=pltpu.VMEM),
)


@jax.jit
def kernel(x, edge_index, W, b, eps, gamma, beta):
    xl = x.reshape(NC * N, DH)
    src = edge_index[0]
    dst = edge_index[1]
    src_even = src * 2
    src_odd = src_even + 1
    zeros = jnp.zeros((RPT, DH), jnp.float32)
    agg = _sc_aggregate(xl, src_even, src_odd, dst, zeros)
    return _tc_finish(x, agg, W, b, eps.reshape(1), gamma, beta)
